# baseline (device time: 183303 ns/iter reference)
import functools

import jax
import jax.numpy as jnp
from jax import lax
from jax.experimental import pallas as pl
from jax.experimental.pallas import tpu as pltpu

N_DEV = 32
LOG2_DEV = 5
N_TOK = 1024
D_IN = 256
D_OUT = 512
E_LOCAL = 4
CAP = 6


def kernel(x, router_W, route_idx, expert_W):
    def body(x_ref, rw_ref, idx_ref, w_ref, out_ref,
             recv_ref, send_sems, recv_sems):
        my_pos = lax.axis_index("i")

        barrier_sem = pltpu.get_barrier_semaphore()
        for k in range(LOG2_DEV):
            partner = my_pos ^ (1 << k)
            pl.semaphore_signal(barrier_sem, inc=1, device_id=(partner,),
                                device_id_type=pl.DeviceIdType.MESH)
        pl.semaphore_wait(barrier_sem, LOG2_DEV)

        e = idx_ref[:, :]
        local_ids = my_pos * E_LOCAL + lax.broadcasted_iota(
            jnp.int32, (1, E_LOCAL), 1)
        match = (e == local_ids).astype(jnp.float32)
        row = lax.broadcasted_iota(jnp.int32, (N_TOK, N_TOK), 0)
        col = lax.broadcasted_iota(jnp.int32, (N_TOK, N_TOK), 1)
        tri = (col < row).astype(jnp.float32)
        prior = jnp.dot(tri, match, preferred_element_type=jnp.float32)
        keep = match * (prior < CAP).astype(jnp.float32)

        xf = x_ref[:, :]
        acc = jnp.zeros((N_TOK, D_OUT), jnp.float32)
        for j in range(E_LOCAL):
            xm = xf * keep[:, j:j + 1]
            acc = acc + jnp.dot(xm, w_ref[j],
                                preferred_element_type=jnp.float32)
        out_ref[:, :] = acc

        for k in range(LOG2_DEV):
            partner = my_pos ^ (1 << k)
            rdma = pltpu.make_async_remote_copy(
                src_ref=out_ref,
                dst_ref=recv_ref.at[k],
                send_sem=send_sems.at[k],
                recv_sem=recv_sems.at[k],
                device_id=(partner,),
                device_id_type=pl.DeviceIdType.MESH,
            )
            rdma.start()
            rdma.wait()
            out_ref[:, :] = out_ref[:, :] + recv_ref[k]

        @functools.partial(pl.run_scoped,
                           exit_sem=pltpu.SemaphoreType.REGULAR)
        def _(exit_sem):
            for k in range(LOG2_DEV):
                partner = my_pos ^ (1 << k)
                pl.semaphore_signal(exit_sem, inc=1, device_id=(partner,),
                                    device_id_type=pl.DeviceIdType.MESH)
            pl.semaphore_wait(exit_sem, LOG2_DEV)

    return pl.pallas_call(
        body,
        out_shape=jax.ShapeDtypeStruct((N_TOK, D_OUT), jnp.float32),
        in_specs=[
            pl.BlockSpec(memory_space=pltpu.VMEM),
            pl.BlockSpec(memory_space=pltpu.VMEM),
            pl.BlockSpec(memory_space=pltpu.VMEM),
            pl.BlockSpec(memory_space=pltpu.VMEM),
        ],
        out_specs=pl.BlockSpec(memory_space=pltpu.VMEM),
        scratch_shapes=[
            pltpu.VMEM((LOG2_DEV, N_TOK, D_OUT), jnp.float32),
            pltpu.SemaphoreType.DMA((LOG2_DEV,)),
            pltpu.SemaphoreType.DMA((LOG2_DEV,)),
        ],
        compiler_params=pltpu.CompilerParams(collective_id=0),
    )(x, router_W, route_idx, expert_W)


# device time: 83082 ns/iter; 2.2063x vs baseline; 2.2063x over previous
import functools

import jax
import jax.numpy as jnp
from jax import lax
from jax.experimental import pallas as pl
from jax.experimental.pallas import tpu as pltpu

N_DEV = 32
LOG2_DEV = 5
N_TOK = 1024
D_IN = 256
D_OUT = 512
E_LOCAL = 4
CAP = 6


def kernel(x, router_W, route_idx, expert_W):
    def body(x_ref, rw_ref, idx_ref, w_ref, out_ref,
             recv_ref, send_sems, recv_sems):
        my_pos = lax.axis_index("i")

        barrier_sem = pltpu.get_barrier_semaphore()
        for k in range(LOG2_DEV):
            partner = my_pos ^ (1 << k)
            pl.semaphore_signal(barrier_sem, inc=1, device_id=(partner,),
                                device_id_type=pl.DeviceIdType.MESH)
        pl.semaphore_wait(barrier_sem, LOG2_DEV)

        e = idx_ref[:, :]
        local_ids = my_pos * E_LOCAL + lax.broadcasted_iota(
            jnp.int32, (1, E_LOCAL), 1)
        match = (e == local_ids).astype(jnp.float32)
        row = lax.broadcasted_iota(jnp.int32, (N_TOK, N_TOK), 0)
        col = lax.broadcasted_iota(jnp.int32, (N_TOK, N_TOK), 1)
        tri = (col < row).astype(jnp.float32)
        prior = jnp.dot(tri, match, preferred_element_type=jnp.float32)
        keep = match * (prior < CAP).astype(jnp.float32)

        xf = x_ref[:, :]
        acc = jnp.zeros((N_TOK, D_OUT), jnp.float32)
        for j in range(E_LOCAL):
            xm = xf * keep[:, j:j + 1]
            acc = acc + jnp.dot(xm, w_ref[j],
                                preferred_element_type=jnp.float32)
        out_ref[:, :] = acc

        seg_start = jnp.int32(0)
        for k in range(LOG2_DEV):
            half = N_TOK >> (k + 1)
            partner = my_pos ^ (1 << k)
            bit = (my_pos >> k) & 1
            send_start = seg_start + (1 - bit) * half
            keep_start = seg_start + bit * half
            rdma = pltpu.make_async_remote_copy(
                src_ref=out_ref.at[pl.ds(send_start, half), :],
                dst_ref=recv_ref.at[k, pl.ds(0, half), :],
                send_sem=send_sems.at[k],
                recv_sem=recv_sems.at[k],
                device_id=(partner,),
                device_id_type=pl.DeviceIdType.MESH,
            )
            rdma.start()
            rdma.wait()
            out_ref[pl.ds(keep_start, half), :] = (
                out_ref[pl.ds(keep_start, half), :] + recv_ref[k, 0:half, :]
            )
            seg_start = keep_start

        cur_start = seg_start
        for k in reversed(range(LOG2_DEV)):
            cur_len = N_TOK >> (k + 1)
            partner = my_pos ^ (1 << k)
            bit = (my_pos >> k) & 1
            idx = LOG2_DEV + (LOG2_DEV - 1 - k)
            rdma = pltpu.make_async_remote_copy(
                src_ref=out_ref.at[pl.ds(cur_start, cur_len), :],
                dst_ref=out_ref.at[pl.ds(cur_start, cur_len), :],
                send_sem=send_sems.at[idx],
                recv_sem=recv_sems.at[idx],
                device_id=(partner,),
                device_id_type=pl.DeviceIdType.MESH,
            )
            rdma.start()
            rdma.wait()
            cur_start = cur_start - bit * cur_len

        @functools.partial(pl.run_scoped,
                           exit_sem=pltpu.SemaphoreType.REGULAR)
        def _(exit_sem):
            for k in range(LOG2_DEV):
                partner = my_pos ^ (1 << k)
                pl.semaphore_signal(exit_sem, inc=1, device_id=(partner,),
                                    device_id_type=pl.DeviceIdType.MESH)
            pl.semaphore_wait(exit_sem, LOG2_DEV)

    return pl.pallas_call(
        body,
        out_shape=jax.ShapeDtypeStruct((N_TOK, D_OUT), jnp.float32),
        in_specs=[
            pl.BlockSpec(memory_space=pltpu.VMEM),
            pl.BlockSpec(memory_space=pltpu.VMEM),
            pl.BlockSpec(memory_space=pltpu.VMEM),
            pl.BlockSpec(memory_space=pltpu.VMEM),
        ],
        out_specs=pl.BlockSpec(memory_space=pltpu.VMEM),
        scratch_shapes=[
            pltpu.VMEM((LOG2_DEV, N_TOK // 2, D_OUT), jnp.float32),
            pltpu.SemaphoreType.DMA((2 * LOG2_DEV,)),
            pltpu.SemaphoreType.DMA((2 * LOG2_DEV,)),
        ],
        compiler_params=pltpu.CompilerParams(collective_id=0),
    )(x, router_W, route_idx, expert_W)


# device time: 25568 ns/iter; 7.1692x vs baseline; 3.2495x over previous
import jax
import jax.numpy as jnp
from jax import lax
from jax.experimental import pallas as pl
from jax.experimental.pallas import tpu as pltpu

N_DEV = 32
N_TOK = 1024
D_IN = 256
D_OUT = 512
E_LOCAL = 4
CAP = 6
BLK = E_LOCAL * CAP
N_SLOT = N_DEV * BLK
HALF_D = N_DEV // 2


def kernel(x, router_W, route_idx, expert_W):
    def body(x_ref, rw_ref, idx_ref, w_ref, out_ref,
             gath_ref, send_sems, recv_sems):
        my_pos = lax.axis_index("i")

        barrier_sem = pltpu.get_barrier_semaphore()
        for d in range(1, N_DEV):
            peer = lax.rem(my_pos + d, N_DEV)
            pl.semaphore_signal(barrier_sem, inc=1, device_id=(peer,),
                                device_id_type=pl.DeviceIdType.MESH)

        e = idx_ref[:, :]
        row = lax.broadcasted_iota(jnp.int32, (N_TOK, N_TOK), 0)
        col = lax.broadcasted_iota(jnp.int32, (N_TOK, N_TOK), 1)
        same = (col < row) & (e == jnp.transpose(e))
        pr = jnp.sum(same.astype(jnp.float32), axis=1, keepdims=True)

        slot24 = lax.broadcasted_iota(jnp.int32, (1, BLK), 1)
        slot_e = my_pos * E_LOCAL + slot24 // CAP
        slot_c = slot24 % CAP
        sel = ((e == slot_e) & (pr == slot_c.astype(jnp.float32)))
        sel = sel.astype(jnp.float32)
        xc = lax.dot_general(sel, x_ref[:, :], (((0,), (0,)), ((), ())),
                             preferred_element_type=jnp.float32)
        yc = jnp.concatenate(
            [jnp.dot(xc[j * CAP:(j + 1) * CAP], w_ref[j],
                     preferred_element_type=jnp.float32)
             for j in range(E_LOCAL)], axis=0)
        gath_ref[0:BLK, :] = yc.astype(jnp.bfloat16)

        pl.semaphore_wait(barrier_sem, N_DEV - 1)

        sends = []
        for d in range(1, N_DEV):
            peer = lax.rem(my_pos + d, N_DEV)
            rdma = pltpu.make_async_remote_copy(
                src_ref=gath_ref.at[0:BLK, :],
                dst_ref=gath_ref.at[d * BLK:(d + 1) * BLK, :],
                send_sem=send_sems.at[d],
                recv_sem=recv_sems.at[d],
                device_id=(peer,),
                device_id_type=pl.DeviceIdType.MESH,
            )
            rdma.start()
            sends.append(rdma)

        slot_g = lax.broadcasted_iota(jnp.int32, (1, N_SLOT), 1)
        rank_g = lax.rem(my_pos - slot_g // BLK + N_DEV, N_DEV)
        eg = rank_g * E_LOCAL + (slot_g % BLK) // CAP
        cg = slot_g % CAP
        P = (e == eg) & (pr == cg.astype(jnp.float32))
        P = P.astype(jnp.bfloat16)

        def wait_d(d):
            recv = pltpu.make_async_remote_copy(
                src_ref=gath_ref.at[d * BLK:(d + 1) * BLK, :],
                dst_ref=gath_ref.at[d * BLK:(d + 1) * BLK, :],
                send_sem=send_sems.at[d],
                recv_sem=recv_sems.at[d],
                device_id=(d,),
                device_id_type=pl.DeviceIdType.MESH,
            )
            recv.wait_recv()

        for d in range(1, HALF_D):
            wait_d(d)
        out_ref[:, :] = jnp.dot(P[:, :HALF_D * BLK],
                                gath_ref[0:HALF_D * BLK, :],
                                preferred_element_type=jnp.float32)

        for d in range(HALF_D, N_DEV):
            wait_d(d)
        out_ref[:, :] = out_ref[:, :] + jnp.dot(
            P[:, HALF_D * BLK:], gath_ref[HALF_D * BLK:N_SLOT, :],
            preferred_element_type=jnp.float32)

        for rdma in sends:
            rdma.wait_send()

    return pl.pallas_call(
        body,
        out_shape=jax.ShapeDtypeStruct((N_TOK, D_OUT), jnp.float32),
        in_specs=[
            pl.BlockSpec(memory_space=pltpu.VMEM),
            pl.BlockSpec(memory_space=pltpu.VMEM),
            pl.BlockSpec(memory_space=pltpu.VMEM),
            pl.BlockSpec(memory_space=pltpu.VMEM),
        ],
        out_specs=pl.BlockSpec(memory_space=pltpu.VMEM),
        scratch_shapes=[
            pltpu.VMEM((N_SLOT, D_OUT), jnp.bfloat16),
            pltpu.SemaphoreType.DMA((N_DEV,)),
            pltpu.SemaphoreType.DMA((N_DEV,)),
        ],
        compiler_params=pltpu.CompilerParams(collective_id=0),
    )(x, router_W, route_idx, expert_W)
